# Initial kernel scaffold; baseline (speedup 1.0000x reference)
import jax
import jax.numpy as jnp
from jax.experimental import pallas as pl


def kernel(pred, target, batch, pos):
    raise NotImplementedError("write your pallas kernel here")


# speedup vs baseline: 15.1863x; 15.1863x over previous
import functools

import jax
import jax.numpy as jnp
from jax import lax
from jax.experimental import pallas as pl
from jax.experimental.pallas import tpu as pltpu
from jax.experimental.pallas import tpu_sc as plsc

N = 1_600_000
SEG = 1024
NF = 8

_info = plsc.get_sparse_core_info()
NC, NS, L = _info.num_cores, _info.num_subcores, _info.num_lanes
NW = NC * NS
PER_W = N // NW
CHUNK = 2000
NCHUNK = PER_W // CHUNK
GROUPS = CHUNK // 16

_mesh = plsc.VectorSubcoreMesh(core_axis_name="c", subcore_axis_name="s")


@functools.partial(
    pl.kernel,
    out_type=jax.ShapeDtypeStruct((NW, NF * SEG), jnp.float32),
    mesh=_mesh,
    compiler_params=pltpu.CompilerParams(needs_layout_passes=False),
    scratch_types=[
        pltpu.VMEM((CHUNK,), jnp.float32),
        pltpu.VMEM((CHUNK,), jnp.float32),
        pltpu.VMEM((CHUNK,), jnp.int32),
        pltpu.VMEM((CHUNK,), jnp.int32),
        pltpu.VMEM((CHUNK,), jnp.int32),
        pltpu.VMEM((CHUNK,), jnp.int32),
        pltpu.VMEM((CHUNK,), jnp.float32),
        pltpu.VMEM((CHUNK,), jnp.float32),
        pltpu.VMEM((CHUNK,), jnp.float32),
        pltpu.VMEM((CHUNK,), jnp.float32),
        pltpu.VMEM((CHUNK,), jnp.float32),
        pltpu.VMEM((CHUNK,), jnp.float32),
        pltpu.VMEM((NF * SEG,), jnp.float32),
        pltpu.SemaphoreType.DMA,
        pltpu.SemaphoreType.DMA,
    ],
)
def _sc_partials(pred_h, targ_h, batch_h, px_h, py_h, pz_h, out_h,
                 pred_b0, pred_b1, targ_b0, targ_b1, batch_b0, batch_b1,
                 px_b0, px_b1, py_b0, py_b1, pz_b0, pz_b1,
                 acc, sem0, sem1):
    wid = lax.axis_index("s") * NC + lax.axis_index("c")
    base = wid * PER_W
    sems = (sem0, sem1)
    bufs = (
        (pred_b0, targ_b0, batch_b0, px_b0, py_b0, pz_b0),
        (pred_b1, targ_b1, batch_b1, px_b1, py_b1, pz_b1),
    )
    hbm = (pred_h, targ_h, batch_h, px_h, py_h, pz_h)

    zero16 = jnp.zeros((16,), jnp.float32)

    def zbody(i, carry):
        acc[pl.ds(i * 16, 16)] = zero16
        return carry

    lax.fori_loop(0, NF * SEG // 16, zbody, 0)

    def start_chunk(k, slot):
        off = base + k * CHUNK
        sem = sems[slot]
        for h, b in zip(hbm, bufs[slot]):
            pltpu.make_async_copy(h.at[pl.ds(off, CHUNK)], b, sem).start()

    def wait_chunk(k, slot):
        off = base + k * CHUNK
        sem = sems[slot]
        for h, b in zip(hbm, bufs[slot]):
            pltpu.make_async_copy(h.at[pl.ds(off, CHUNK)], b, sem).wait()

    onef = jnp.full((16,), 1.0, jnp.float32)
    zerof = jnp.zeros((16,), jnp.float32)

    def make_group(slot):
        pb, tb, bb, xb, yb, zb = bufs[slot]

        def group(g, carry):
            o = g * 16
            p = pb[pl.ds(o, 16)]
            t = tb[pl.ds(o, 16)]
            b = bb[pl.ds(o, 16)]
            x = xb[pl.ds(o, 16)]
            y = yb[pl.ds(o, 16)]
            z = zb[pl.ds(o, 16)]
            wt = jnp.where(t == 1, onef, zerof)
            plsc.addupdate_scatter(acc, [b], wt * x)
            plsc.addupdate_scatter(acc, [b + SEG], wt * y)
            plsc.addupdate_scatter(acc, [b + 2 * SEG], wt * z)
            plsc.addupdate_scatter(acc, [b + 3 * SEG], wt)
            plsc.addupdate_scatter(acc, [b + 4 * SEG], p * x)
            plsc.addupdate_scatter(acc, [b + 5 * SEG], p * y)
            plsc.addupdate_scatter(acc, [b + 6 * SEG], p * z)
            plsc.addupdate_scatter(acc, [b + 7 * SEG], p)
            return carry

        return group

    group_fns = (make_group(0), make_group(1))

    start_chunk(0, 0)
    for k in range(NCHUNK):
        slot = k % 2
        if k + 1 < NCHUNK:
            start_chunk(k + 1, (k + 1) % 2)
        wait_chunk(k, slot)
        lax.fori_loop(0, GROUPS, group_fns[slot], 0)

    pltpu.sync_copy(acc, out_h.at[wid])


def _finish_body(p_ref, o_ref):
    a = jnp.sum(p_ref[...], axis=0)
    eps = jnp.float32(1e-10)
    tc = a[0:3, :] / (a[3:4, :] + eps)
    pc = a[4:7, :] / (a[7:8, :] + eps)
    d = tc - pc
    o_ref[0, 0] = jnp.sqrt(jnp.sum(d * d))


_finish = pl.pallas_call(
    _finish_body,
    out_shape=jax.ShapeDtypeStruct((1, 1), jnp.float32),
    out_specs=pl.BlockSpec(memory_space=pltpu.SMEM),
)


def kernel(pred, target, batch, pos):
    px = pos[:, 0]
    py = pos[:, 1]
    pz = pos[:, 2]
    partials = _sc_partials(pred, target, batch, px, py, pz)
    loss = _finish(partials.reshape(NW, NF, SEG))
    return loss[0, 0]
